# baseline (speedup 1.0000x reference)
import functools

import jax
import jax.numpy as jnp
from jax import lax
from jax.experimental import pallas as pl
from jax.experimental.pallas import tpu as pltpu
from jax.experimental.pallas import tpu_sc as plsc

NUM_USERS = 100000
NUM_ITEMS = 100000
D = 128
B = 16384

NC, NS, L = 2, 16, 16
NW = NC * NS
BPW = B // NW
CHUNK = 128
NCHUNK = BPW // CHUNK
UNROLL = 16


def _mf_body(uidx_hbm, iidx_hbm, uemb_hbm, iemb_hbm, ubias_hbm, ibias_hbm,
             ob_hbm, out_hbm,
             uidx_v, iidx_v, ub_v, ib_v, ob_v, out_v, u_rows, i_rows,
             u_sem0, u_sem1, i_sem0, i_sem1, b_sem):
    wid = lax.axis_index("s") * NC + lax.axis_index("c")
    base = wid * BPW

    pltpu.sync_copy(uidx_hbm.at[pl.ds(base, BPW)], uidx_v)
    pltpu.sync_copy(iidx_hbm.at[pl.ds(base, BPW)], iidx_v)
    pltpu.sync_copy(ob_hbm, ob_v)

    u_sems = (u_sem0, u_sem1)
    i_sems = (i_sem0, i_sem1)

    def fire(c):
        p = c % 2
        sl = pl.ds(c * CHUNK, CHUNK)
        cu = pltpu.async_copy(uemb_hbm.at[uidx_v.at[sl]], u_rows.at[p], u_sems[p])
        ci = pltpu.async_copy(iemb_hbm.at[iidx_v.at[sl]], i_rows.at[p], i_sems[p])
        return cu, ci

    copies = [fire(0)]

    bias_copies = []
    for c in range(NCHUNK):
        sl = pl.ds(c * CHUNK, CHUNK)
        bias_copies.append(pltpu.async_copy(ubias_hbm.at[uidx_v.at[sl]], ub_v.at[sl], b_sem))
        bias_copies.append(pltpu.async_copy(ibias_hbm.at[iidx_v.at[sl]], ib_v.at[sl], b_sem))
    for bc in bias_copies:
        bc.wait()
    obv = ob_v[...]

    for c in range(NCHUNK):
        if c + 1 < NCHUNK:
            copies.append(fire(c + 1))
        cu, ci = copies[c]
        cu.wait()
        ci.wait()
        p = c % 2
        lane = lax.iota(jnp.int32, L)

        for g in range(CHUNK // L):
            rows = lane + (g * L)

            def jbody(jj, acc, rows=rows, p=p):
                col0 = jnp.full((L,), jj * UNROLL, dtype=jnp.int32)
                for k in range(UNROLL):
                    col = col0 + k
                    uv = plsc.load_gather(u_rows.at[p], [rows, col])
                    iv = plsc.load_gather(i_rows.at[p], [rows, col])
                    acc = acc + uv * iv
                return acc

            acc = lax.fori_loop(0, D // UNROLL, jbody,
                                jnp.zeros((L,), jnp.float32))
            sl16 = pl.ds(c * CHUNK + g * L, L)
            out_v[sl16] = acc + ub_v[sl16] + ib_v[sl16] + obv

    pltpu.sync_copy(out_v, out_hbm.at[pl.ds(base, BPW)])


@jax.jit
def kernel(userIdx, itemIdx, uEmbd, iEmbd, uBias, iBias, overAllBias):
    mesh = plsc.VectorSubcoreMesh(core_axis_name="c", subcore_axis_name="s")
    mf = functools.partial(
        pl.kernel,
        out_type=jax.ShapeDtypeStruct((B,), jnp.float32),
        mesh=mesh,
        compiler_params=pltpu.CompilerParams(needs_layout_passes=False),
        scratch_types=[
            pltpu.VMEM((BPW,), jnp.int32),
            pltpu.VMEM((BPW,), jnp.int32),
            pltpu.VMEM((BPW,), jnp.float32),
            pltpu.VMEM((BPW,), jnp.float32),
            pltpu.VMEM((L,), jnp.float32),
            pltpu.VMEM((BPW,), jnp.float32),
            pltpu.VMEM((2, CHUNK, D), jnp.float32),
            pltpu.VMEM((2, CHUNK, D), jnp.float32),
            pltpu.SemaphoreType.DMA,
            pltpu.SemaphoreType.DMA,
            pltpu.SemaphoreType.DMA,
            pltpu.SemaphoreType.DMA,
            pltpu.SemaphoreType.DMA,
        ],
    )(_mf_body)
    ob16 = jnp.broadcast_to(overAllBias.astype(jnp.float32), (L,))
    return mf(userIdx.astype(jnp.int32), itemIdx.astype(jnp.int32),
              uEmbd, iEmbd,
              uBias.reshape(NUM_USERS), iBias.reshape(NUM_ITEMS), ob16)


# speedup vs baseline: 2.3367x; 2.3367x over previous
import functools

import jax
import jax.numpy as jnp
from jax import lax
from jax.experimental import pallas as pl
from jax.experimental.pallas import tpu as pltpu
from jax.experimental.pallas import tpu_sc as plsc

NUM_USERS = 100000
NUM_ITEMS = 100000
D = 128
B = 16384

NC, NS, L = 2, 16, 16
NW = NC * NS
BPW = B // NW
CHUNK = 128
NCHUNK = BPW // CHUNK
UNROLL = 16


def _mf_body(uidx_hbm, iidx_hbm, uemb_hbm, iemb_hbm, ubias_hbm, ibias_hbm,
             ob_hbm, out_hbm,
             uidx_v, iidx_v, ub_v, ib_v, ob_v, out_v, u_rows, i_rows, tp_v,
             u_sem0, u_sem1, i_sem0, i_sem1, b_sem):
    wid = lax.axis_index("s") * NC + lax.axis_index("c")
    base = wid * BPW

    pltpu.sync_copy(uidx_hbm.at[pl.ds(base, BPW)], uidx_v)
    pltpu.sync_copy(iidx_hbm.at[pl.ds(base, BPW)], iidx_v)
    pltpu.sync_copy(ob_hbm, ob_v)

    u_sems = (u_sem0, u_sem1)
    i_sems = (i_sem0, i_sem1)

    def fire(c):
        p = c % 2
        sl = pl.ds(c * CHUNK, CHUNK)
        cu = pltpu.async_copy(uemb_hbm.at[uidx_v.at[sl]], u_rows.at[p], u_sems[p])
        ci = pltpu.async_copy(iemb_hbm.at[iidx_v.at[sl]], i_rows.at[p], i_sems[p])
        return cu, ci

    copies = [fire(0)]

    bias_copies = []
    for c in range(NCHUNK):
        sl = pl.ds(c * CHUNK, CHUNK)
        bias_copies.append(pltpu.async_copy(ubias_hbm.at[uidx_v.at[sl]], ub_v.at[sl], b_sem))
        bias_copies.append(pltpu.async_copy(ibias_hbm.at[iidx_v.at[sl]], ib_v.at[sl], b_sem))
    for bc in bias_copies:
        bc.wait()
    obv = ob_v[...]

    for c in range(NCHUNK):
        if c + 1 < NCHUNK:
            copies.append(fire(c + 1))
        cu, ci = copies[c]
        cu.wait()
        ci.wait()
        p = c % 2
        lane = lax.iota(jnp.int32, L)

        def gbody(g, _, p=p, c=c):
            for rr in range(L):
                r = g * L + rr
                sl0 = pl.ds(0, L)
                acc = u_rows[p, r, sl0] * i_rows[p, r, sl0]
                for k in range(1, D // L):
                    sl = pl.ds(k * L, L)
                    acc = acc + u_rows[p, r, sl] * i_rows[p, r, sl]
                tp_v[rr, sl0] = acc
            s = plsc.load_gather(tp_v, [lane, jnp.zeros((L,), jnp.int32)])
            for l in range(1, L):
                s = s + plsc.load_gather(tp_v, [lane, jnp.full((L,), l, jnp.int32)])
            sl16 = pl.ds(c * CHUNK + g * L, L)
            out_v[sl16] = s + ub_v[sl16] + ib_v[sl16] + obv
            return _

        lax.fori_loop(0, CHUNK // L, gbody, 0)

    pltpu.sync_copy(out_v, out_hbm.at[pl.ds(base, BPW)])


@jax.jit
def kernel(userIdx, itemIdx, uEmbd, iEmbd, uBias, iBias, overAllBias):
    mesh = plsc.VectorSubcoreMesh(core_axis_name="c", subcore_axis_name="s")
    mf = functools.partial(
        pl.kernel,
        out_type=jax.ShapeDtypeStruct((B,), jnp.float32),
        mesh=mesh,
        compiler_params=pltpu.CompilerParams(needs_layout_passes=False),
        scratch_types=[
            pltpu.VMEM((BPW,), jnp.int32),
            pltpu.VMEM((BPW,), jnp.int32),
            pltpu.VMEM((BPW,), jnp.float32),
            pltpu.VMEM((BPW,), jnp.float32),
            pltpu.VMEM((L,), jnp.float32),
            pltpu.VMEM((BPW,), jnp.float32),
            pltpu.VMEM((2, CHUNK, D), jnp.float32),
            pltpu.VMEM((2, CHUNK, D), jnp.float32),
            pltpu.VMEM((L, L + 1), jnp.float32),
            pltpu.SemaphoreType.DMA,
            pltpu.SemaphoreType.DMA,
            pltpu.SemaphoreType.DMA,
            pltpu.SemaphoreType.DMA,
            pltpu.SemaphoreType.DMA,
        ],
    )(_mf_body)
    ob16 = jnp.broadcast_to(overAllBias.astype(jnp.float32), (L,))
    return mf(userIdx.astype(jnp.int32), itemIdx.astype(jnp.int32),
              uEmbd, iEmbd,
              uBias.reshape(NUM_USERS), iBias.reshape(NUM_ITEMS), ob16)
